# Initial kernel scaffold; baseline (speedup 1.0000x reference)
import jax
import jax.numpy as jnp
from jax.experimental import pallas as pl


def kernel(inputs, table, W1, b1, W2, b2):
    raise NotImplementedError("write your pallas kernel here")


# speedup vs baseline: 1.5216x; 1.5216x over previous
import functools

import jax
import jax.numpy as jnp
from jax import lax
from jax.experimental import pallas as pl
from jax.experimental.pallas import tpu as pltpu
from jax.experimental.pallas import tpu_sc as plsc

BATCH = 4096
SEQ = 200
SEQ_PAD = 208
HALF = SEQ_PAD // 2
EMB = 32
HID = 128
OUT = 64

NUM_WORKERS = 32
ROWS_PER_W = BATCH // NUM_WORKERS
HALVES_PER_W = 2 * ROWS_PER_W
NBUF = 4
GROUPS = HALVES_PER_W // NBUF


def _pool_body(table_hbm, idx_hbm, out_hbm, idx_v, b0, b1, b2, b3, outs_v,
               s0, s1, s2, s3):
    bufs = (b0, b1, b2, b3)
    sems = (s0, s1, s2, s3)
    wid = lax.axis_index("s") * 2 + lax.axis_index("c")
    base_half = wid * HALVES_PER_W
    base_row = wid * ROWS_PER_W

    pltpu.sync_copy(idx_hbm.at[pl.ds(base_half, HALVES_PER_W)], idx_v)

    def group(g, carry):
        cps = [
            pltpu.async_copy(table_hbm.at[idx_v.at[NBUF * g + k]],
                             bufs[k], sems[k])
            for k in range(NBUF)
        ]
        for r in range(2):
            acc_lo = jnp.zeros((16,), jnp.float32)
            acc_hi = jnp.zeros((16,), jnp.float32)
            for k in (2 * r, 2 * r + 1):
                cps[k].wait()
                buf = bufs[k]
                for s in range(HALF):
                    acc_lo = acc_lo + buf[s, 0:16]
                    acc_hi = acc_hi + buf[s, 16:32]
            row = 2 * g + r
            outs_v[row, 0:16] = acc_lo
            outs_v[row, 16:32] = acc_hi
        return carry

    lax.fori_loop(0, GROUPS, group, 0)
    pltpu.sync_copy(outs_v, out_hbm.at[pl.ds(base_row, ROWS_PER_W)])


_pooled_sum = functools.partial(
    pl.kernel,
    mesh=plsc.VectorSubcoreMesh(core_axis_name="c", subcore_axis_name="s"),
    compiler_params=pltpu.CompilerParams(use_tc_tiling_on_sc=False),
    out_type=jax.ShapeDtypeStruct((BATCH, EMB), jnp.float32),
    scratch_types=[
        pltpu.VMEM((HALVES_PER_W, HALF), jnp.int32),
        pltpu.VMEM((HALF, EMB), jnp.float32),
        pltpu.VMEM((HALF, EMB), jnp.float32),
        pltpu.VMEM((HALF, EMB), jnp.float32),
        pltpu.VMEM((HALF, EMB), jnp.float32),
        pltpu.VMEM((ROWS_PER_W, EMB), jnp.float32),
        pltpu.SemaphoreType.DMA,
        pltpu.SemaphoreType.DMA,
        pltpu.SemaphoreType.DMA,
        pltpu.SemaphoreType.DMA,
    ],
)(_pool_body)


def _mlp_body(pooled_ref, idx_ref, t0_ref, w1_ref, bb1_ref, w2_ref, bb2_ref,
              out_ref):
    pooled = pooled_ref[...]
    idx = idx_ref[...]
    c0 = jnp.sum((idx == 0).astype(jnp.float32), axis=1, keepdims=True) + 8.0
    x = (pooled - c0 * t0_ref[...]) * (1.0 / SEQ)
    h = jnp.dot(x, w1_ref[...], preferred_element_type=jnp.float32,
                precision=lax.Precision.HIGHEST) + bb1_ref[...]
    z = jnp.dot(h, w2_ref[...], preferred_element_type=jnp.float32,
                precision=lax.Precision.HIGHEST) + bb2_ref[...]
    z = z - jnp.max(z, axis=1, keepdims=True)
    e = jnp.exp(z)
    out_ref[...] = e / jnp.sum(e, axis=1, keepdims=True)


def _mlp_call(pooled, idx, t0, w1, bb1, w2, bb2):
    bt = 512
    grid = (BATCH // bt,)
    return pl.pallas_call(
        _mlp_body,
        out_shape=jax.ShapeDtypeStruct((BATCH, OUT), jnp.float32),
        grid=grid,
        in_specs=[
            pl.BlockSpec((bt, EMB), lambda i: (i, 0)),
            pl.BlockSpec((bt, SEQ), lambda i: (i, 0)),
            pl.BlockSpec((1, EMB), lambda i: (0, 0)),
            pl.BlockSpec((EMB, HID), lambda i: (0, 0)),
            pl.BlockSpec((1, HID), lambda i: (0, 0)),
            pl.BlockSpec((HID, OUT), lambda i: (0, 0)),
            pl.BlockSpec((1, OUT), lambda i: (0, 0)),
        ],
        out_specs=pl.BlockSpec((bt, OUT), lambda i: (i, 0)),
    )(pooled, idx, t0, w1, bb1, w2, bb2)


def kernel(inputs, table, W1, b1, W2, b2):
    idx = inputs.astype(jnp.int32)
    idx_pad = jnp.pad(idx, ((0, 0), (0, SEQ_PAD - SEQ)))
    idx_halves = idx_pad.reshape(BATCH * 2, HALF)
    pooled = _pooled_sum(table, idx_halves)
    t0 = table[0:1]
    return _mlp_call(pooled, idx, t0, W1, b1.reshape(1, HID), W2,
                     b2.reshape(1, OUT))
